# baseline (speedup 1.0000x reference)
import functools

import jax
import jax.numpy as jnp
from jax import lax
from jax.experimental import pallas as pl
from jax.experimental.pallas import tpu as pltpu
from jax.experimental.pallas import tpu_sc as plsc

DIM = 256
K = 1024
N = 16 * 576
NCHUNK = 1
CN = N // NCHUNK
TN = 2304
NSTEP = N // TN

NC = 2
NS = 16
NW = NC * NS
B_PER_W = CN // NW
CHUNK = 96
N_STREAM = B_PER_W // CHUNK


def _argmin_body(x_ref, e_ref, idx_ref):
    x = x_ref[...]
    e = e_ref[...]
    s2 = lax.dot_general(e, x * (-2.0), (((1,), (1,)), ((), ())),
                         preferred_element_type=jnp.float32)
    xnorm = jnp.transpose(jnp.sum(x * x, axis=1, keepdims=True))
    enorm = jnp.sum(e * e, axis=1)[:, None]
    u = (xnorm + s2) + enorm
    m = jnp.min(u, axis=0)[None, :]
    iota = lax.broadcasted_iota(jnp.int32, (K, 1), 0).astype(jnp.float32)
    penal = jnp.where(u == m, iota, float(K))
    idx_ref[...] = jnp.min(penal, axis=0).astype(jnp.int32).reshape(1, 1, TN)


def _sc_gather_body(e_hbm, idx_hbm, out_hbm, idx_v, rows_v, gsem, wsem):
    wid = lax.axis_index("s") * NC + lax.axis_index("c")
    base = wid * B_PER_W
    pltpu.sync_copy(idx_hbm.at[pl.ds(base, B_PER_W)], idx_v)
    gathers = [
        pltpu.async_copy(
            e_hbm.at[idx_v.at[pl.ds(c * CHUNK, CHUNK)]],
            rows_v.at[pl.ds(c * CHUNK, CHUNK)], gsem)
        for c in range(N_STREAM)
    ]
    writes = []
    for c in range(N_STREAM):
        gathers[c].wait()
        writes.append(pltpu.async_copy(
            rows_v.at[pl.ds(c * CHUNK, CHUNK)],
            out_hbm.at[pl.ds(base + c * CHUNK, CHUNK)], wsem))
    for w in writes:
        w.wait()


def _make_sc_gather():
    return pl.kernel(
        _sc_gather_body,
        mesh=plsc.VectorSubcoreMesh(core_axis_name="c", subcore_axis_name="s"),
        out_type=jax.ShapeDtypeStruct((CN, DIM), jnp.float32),
        scratch_types=[
            pltpu.VMEM((B_PER_W,), jnp.int32),
            pltpu.VMEM((B_PER_W, DIM), jnp.float32),
            pltpu.SemaphoreType.DMA,
            pltpu.SemaphoreType.DMA,
        ],
    )


def _tc_argmin(flat_chunk, embed):
    return pl.pallas_call(
        _argmin_body,
        grid=(CN // TN,),
        in_specs=[
            pl.BlockSpec((TN, DIM), lambda i: (i, 0)),
            pl.BlockSpec((K, DIM), lambda i: (0, 0)),
        ],
        out_specs=pl.BlockSpec((1, 1, TN), lambda i: (i, 0, 0)),
        out_shape=jax.ShapeDtypeStruct((NSTEP, 1, TN), jnp.int32),
    )(flat_chunk, embed).reshape(N)


@jax.jit
def _vq(flat, embed):
    idx = _tc_argmin(flat, embed)
    embed_b, idx_b = lax.optimization_barrier((embed, idx))
    quantize = _make_sc_gather()(embed_b, idx_b)
    return quantize, idx


def kernel(x, embed):
    shape = x.shape
    flat = x.reshape(-1, shape[-1])
    quantize, idx = _vq(flat, embed)
    return quantize.reshape(shape), idx.reshape(shape[:-1])


# speedup vs baseline: 1.0447x; 1.0447x over previous
import jax
import jax.numpy as jnp
from jax import lax
from jax.experimental import pallas as pl
from jax.experimental.pallas import tpu as pltpu
from jax.experimental.pallas import tpu_sc as plsc

DIM = 256
K = 1024
N = 16 * 576
TN = 1024

NC = 2
NS = 16
NW = NC * NS
B_PER_W = N // NW
CHUNK = 96
N_STREAM = B_PER_W // CHUNK


def _argmin_body(x_ref, e_ref, idx_ref):
    x = x_ref[...]
    e = e_ref[...]
    s2 = lax.dot_general(e, x * (-2.0), (((1,), (1,)), ((), ())),
                         preferred_element_type=jnp.float32)
    xnorm = jnp.transpose(jnp.sum(x * x, axis=1, keepdims=True))
    enorm = jnp.sum(e * e, axis=1)[:, None]
    u = (xnorm + s2) + enorm
    m = jnp.min(u, axis=0)[None, :]
    iota = lax.broadcasted_iota(jnp.int32, (K, 1), 0).astype(jnp.float32)
    penal = jnp.where(u == m, iota, float(K))
    idx_ref[...] = jnp.min(penal, axis=0).astype(jnp.int32)


def _sc_gather_body(e_hbm, idx_hbm, out_hbm, idx_v, rows_v, sem):
    wid = lax.axis_index("s") * NC + lax.axis_index("c")
    base = wid * B_PER_W
    pltpu.sync_copy(idx_hbm.at[pl.ds(base, B_PER_W)], idx_v)
    gathers = [
        pltpu.async_copy(
            e_hbm.at[idx_v.at[pl.ds(c * CHUNK, CHUNK)]],
            rows_v.at[pl.ds(c * CHUNK, CHUNK)], sem)
        for c in range(N_STREAM)
    ]
    for g in gathers:
        g.wait()
    pltpu.sync_copy(rows_v, out_hbm.at[pl.ds(base, B_PER_W)])


def _make_sc_gather():
    return pl.kernel(
        _sc_gather_body,
        mesh=plsc.VectorSubcoreMesh(core_axis_name="c", subcore_axis_name="s"),
        out_type=jax.ShapeDtypeStruct((N, DIM), jnp.float32),
        scratch_types=[
            pltpu.VMEM((B_PER_W,), jnp.int32),
            pltpu.VMEM((B_PER_W, DIM), jnp.float32),
            pltpu.SemaphoreType.DMA,
        ],
    )


def _tc_argmin(flat, embed):
    return pl.pallas_call(
        _argmin_body,
        grid=(N // TN,),
        in_specs=[
            pl.BlockSpec((TN, DIM), lambda i: (i, 0)),
            pl.BlockSpec((K, DIM), lambda i: (0, 0)),
        ],
        out_specs=pl.BlockSpec((TN,), lambda i: (i,)),
        out_shape=jax.ShapeDtypeStruct((N,), jnp.int32),
    )(flat, embed)


@jax.jit
def _vq(flat, embed):
    idx = _tc_argmin(flat, embed)
    quantize = _make_sc_gather()(embed, idx)
    return quantize, idx


def kernel(x, embed):
    shape = x.shape
    flat = x.reshape(-1, shape[-1])
    quantize, idx = _vq(flat, embed)
    return quantize.reshape(shape), idx.reshape(shape[:-1])
